# baseline (speedup 1.0000x reference)
import functools

import jax
import jax.numpy as jnp
from jax import lax
from jax.experimental import pallas as pl
from jax.experimental.pallas import tpu as pltpu
from jax.experimental.pallas import tpu_sc as plsc

EMB_DIM = 64
_PARAMS = pltpu.CompilerParams(needs_layout_passes=False)


@functools.lru_cache(maxsize=None)
def _make_format(V: int):
    info = plsc.get_sparse_core_info()
    nw = info.num_cores * info.num_subcores
    nblk = V // 128
    tail = V - nblk * 128
    ngrp = nblk // 4
    assert nblk % 4 == 0
    nslot = ngrp // nw + 1
    npair = (nslot + 1) // 2
    mesh = plsc.VectorSubcoreMesh(core_axis_name="c", subcore_axis_name="s")

    @functools.partial(
        pl.kernel,
        mesh=mesh,
        compiler_params=_PARAMS,
        out_type=jax.ShapeDtypeStruct((V // 2, 128), jnp.float32),
        scratch_types=[
            pltpu.VMEM((2, EMB_DIM, 512), jnp.float32),
            pltpu.VMEM((2, 128, 128), jnp.float32),
            pltpu.VMEM((EMB_DIM, EMB_DIM), jnp.float32),
            pltpu.VMEM((32, 128), jnp.float32),
            [pltpu.SemaphoreType.DMA] * 2,
            [pltpu.SemaphoreType.DMA] * 2,
        ],
    )
    def format_kernel(tt_hbm, t2_hbm, col_v, rows_v, tcol_v, trows_v,
                      lsems, ssems):
        wid = lax.axis_index("s") * info.num_cores + lax.axis_index("c")
        iota16 = lax.iota(jnp.int32, 16)
        rsrc = [(16 * g + iota16) % EMB_DIM for g in range(8)]
        cadd = [(16 * g) // EMB_DIM for g in range(8)]
        ccs = [iota16 + 16 * g for g in range(8)]

        def grp_of(t):
            return t * nw + wid

        def active(t):
            return grp_of(t) < ngrp

        def start_load(t, ln):
            jg = grp_of(t)
            pltpu.async_copy(tt_hbm.at[:, pl.ds(jg * 512, 512)],
                             col_v.at[ln], lsems[ln])

        def wait_load(ln):
            pltpu.make_async_copy(tt_hbm.at[:, pl.ds(0, 512)],
                                  col_v.at[ln], lsems[ln]).wait()

        def transpose_block(col, rows, b, roff):
            def pbody(pp, c):
                r16 = (pp + iota16) & (EMB_DIM - 1)
                csrc = 2 * r16 + 128 * b
                for g in range(8):
                    vals = plsc.load_gather(col, [rsrc[g], csrc + cadd[g]])
                    plsc.store_scatter(rows, [r16 + roff, ccs[g]], vals)
                return c

            lax.fori_loop(0, EMB_DIM, pbody, 0)

        def start_store(t, half, ln):
            jg = grp_of(t)
            pltpu.async_copy(rows_v.at[ln],
                             t2_hbm.at[pl.ds(jg * 256 + 128 * half, 128)],
                             ssems[ln])

        def wait_store(ln):
            pltpu.make_async_copy(rows_v.at[ln], t2_hbm.at[pl.ds(0, 128)],
                                  ssems[ln]).wait()

        start_load(0, 0)
        start_load(1, 1)

        def pair_body(p, carry):
            for ln in range(2):
                t = 2 * p + ln

                @pl.when(active(t))
                def _(t=t, ln=ln):
                    wait_load(ln)
                    for half in range(2):
                        if ln == 0:
                            @pl.when(p > 0)
                            def _(half=half):
                                wait_store(half)
                        else:
                            wait_store(half)

                        transpose_block(col_v.at[ln], rows_v.at[half],
                                        2 * half, 0)
                        transpose_block(col_v.at[ln], rows_v.at[half],
                                        2 * half + 1, EMB_DIM)
                        start_store(t, half, half)

                @pl.when(jnp.logical_and(p + 1 < npair, active(t + 2)))
                def _(t=t, ln=ln):
                    start_load(t + 2, ln)

            return carry

        lax.fori_loop(0, npair, pair_body, 0)
        wait_store(0)
        wait_store(1)

        @pl.when(jnp.logical_and(wid == 0, tail > 0))
        def _():
            pltpu.sync_copy(tt_hbm.at[:, pl.ds(nblk * 128, tail)], tcol_v)

            def pbody(pp, c):
                r16 = (pp + iota16) & (tail // 2 - 1)
                col2 = 2 * r16
                for g in range(8):
                    vals = plsc.load_gather(tcol_v, [rsrc[g], col2 + cadd[g]])
                    plsc.store_scatter(trows_v, [r16, ccs[g]], vals)
                return c

            lax.fori_loop(0, tail // 2, pbody, 0)
            pltpu.sync_copy(trows_v, t2_hbm.at[pl.ds(nblk * 64, tail // 2)])

    return format_kernel


@functools.lru_cache(maxsize=None)
def _make_gather(S: int, B: int, V2: int):
    info = plsc.get_sparse_core_info()
    nw = info.num_cores * info.num_subcores
    assert B == 128 * nw and S % 4 == 0
    nq = S // 4
    mesh = plsc.VectorSubcoreMesh(core_axis_name="c", subcore_axis_name="s")

    @functools.partial(
        pl.kernel,
        mesh=mesh,
        compiler_params=_PARAMS,
        out_type=jax.ShapeDtypeStruct((S, EMB_DIM, B), jnp.float32),
        scratch_types=[
            pltpu.VMEM((S, 128), jnp.int32),
            pltpu.VMEM((4, 128), jnp.int32),
            pltpu.VMEM((4, 128, 128), jnp.float32),
            pltpu.VMEM((4, EMB_DIM, 128), jnp.float32),
            [pltpu.SemaphoreType.DMA] * 4,
            [pltpu.SemaphoreType.DMA] * 4,
        ],
    )
    def gather_kernel(xt_hbm, t2_hbm, out_hbm, idx_v, idx2_v, rows_v,
                      tiles_v, gsems, ssems):
        wid = lax.axis_index("s") * info.num_cores + lax.axis_index("c")
        boff = wid * 128
        iota16 = lax.iota(jnp.int32, 16)
        bbs = tuple(iota16 + 16 * g for g in range(8))

        pltpu.sync_copy(xt_hbm.at[:, pl.ds(boff, 128)], idx_v)

        def start_gather(s, ln):
            for g in range(8):
                idx2_v[ln, pl.ds(16 * g, 16)] = idx_v[s, pl.ds(16 * g, 16)] >> 1
            pltpu.async_copy(t2_hbm.at[idx2_v.at[ln]], rows_v.at[ln],
                             gsems[ln])

        def wait_gather(ln):
            pltpu.make_async_copy(t2_hbm.at[idx2_v.at[ln]], rows_v.at[ln],
                                  gsems[ln]).wait()

        def transpose(s, ln):
            rows = rows_v.at[ln]
            tiles = tiles_v.at[ln]
            bases = tuple(
                (idx_v[s, pl.ds(16 * g, 16)] & 1) * EMB_DIM for g in range(8))

            def dbody(dd, carry):
                d16 = (dd + iota16) & (EMB_DIM - 1)
                for g in range(8):
                    vals = plsc.load_gather(rows, [bbs[g], carry[g] + d16])
                    plsc.store_scatter(tiles, [d16, bbs[g]], vals)
                return carry

            lax.fori_loop(0, EMB_DIM, dbody, bases)

        def start_store(s, ln):
            pltpu.async_copy(tiles_v.at[ln],
                             out_hbm.at[s, :, pl.ds(boff, 128)], ssems[ln])

        def wait_store(ln):
            pltpu.make_async_copy(tiles_v.at[ln],
                                  out_hbm.at[0, :, pl.ds(boff, 128)],
                                  ssems[ln]).wait()

        for ln in range(4):
            start_gather(ln, ln)

        def quad_body(q, carry):
            for ln in range(4):
                s = 4 * q + ln

                @pl.when(q > 0)
                def _(ln=ln):
                    wait_store(ln)

                wait_gather(ln)
                transpose(s, ln)

                @pl.when(q + 1 < nq)
                def _(s=s, ln=ln):
                    start_gather(s + 4, ln)

                start_store(s, ln)
            return carry

        lax.fori_loop(0, nq, quad_body, 0)
        for ln in range(4):
            wait_store(ln)

    return gather_kernel


def kernel(x, table):
    b, s = x.shape
    v, d = table.shape
    t2 = _make_format(v)(table.T)
    out_t = _make_gather(s, b, v // 2)(x.T, t2)
    return out_t.transpose(2, 0, 1)


# speedup vs baseline: 2.3546x; 2.3546x over previous
import functools

import jax
import jax.numpy as jnp
from jax import lax
from jax.experimental import pallas as pl
from jax.experimental.pallas import tpu as pltpu
from jax.experimental.pallas import tpu_sc as plsc

EMB_DIM = 64
_PARAMS = pltpu.CompilerParams(needs_layout_passes=False)


@functools.lru_cache(maxsize=None)
def _make_format(V: int):
    info = plsc.get_sparse_core_info()
    nw = info.num_cores * info.num_subcores
    nblk = V // 128
    tail = V - nblk * 128
    ngrp = nblk // 4
    assert nblk % 4 == 0
    nslot = ngrp // nw + 1
    npair = (nslot + 1) // 2
    mesh = plsc.VectorSubcoreMesh(core_axis_name="c", subcore_axis_name="s")

    @functools.partial(
        pl.kernel,
        mesh=mesh,
        compiler_params=_PARAMS,
        out_type=jax.ShapeDtypeStruct((V // 2, 128), jnp.float32),
        scratch_types=[
            pltpu.VMEM((2, EMB_DIM, 512), jnp.float32),
            pltpu.VMEM((2, 128, 128), jnp.float32),
            pltpu.VMEM((EMB_DIM, EMB_DIM), jnp.float32),
            pltpu.VMEM((32, 128), jnp.float32),
            [pltpu.SemaphoreType.DMA] * 2,
            [pltpu.SemaphoreType.DMA] * 2,
        ],
    )
    def format_kernel(tt_hbm, t2_hbm, col_v, rows_v, tcol_v, trows_v,
                      lsems, ssems):
        wid = lax.axis_index("s") * info.num_cores + lax.axis_index("c")
        iota16 = lax.iota(jnp.int32, 16)
        rsrc = [(16 * g + iota16) % EMB_DIM for g in range(8)]
        cadd = [(16 * g) // EMB_DIM for g in range(8)]
        ccs = [iota16 + 16 * g for g in range(8)]

        def grp_of(t):
            return t * nw + wid

        def active(t):
            return grp_of(t) < ngrp

        def start_load(t, ln):
            jg = grp_of(t)
            pltpu.async_copy(tt_hbm.at[:, pl.ds(jg * 512, 512)],
                             col_v.at[ln], lsems[ln])

        def wait_load(ln):
            pltpu.make_async_copy(tt_hbm.at[:, pl.ds(0, 512)],
                                  col_v.at[ln], lsems[ln]).wait()

        def transpose_block(col, rows, b, roff):
            @functools.partial(plsc.parallel_loop, 0, EMB_DIM, unroll=4)
            def pbody(pp):
                r16 = (pp + iota16) & (EMB_DIM - 1)
                csrc = 2 * r16 + 128 * b
                for g in range(8):
                    vals = plsc.load_gather(col, [rsrc[g], csrc + cadd[g]])
                    plsc.store_scatter(rows, [r16 + roff, ccs[g]], vals)

        def start_store(t, half, ln):
            jg = grp_of(t)
            pltpu.async_copy(rows_v.at[ln],
                             t2_hbm.at[pl.ds(jg * 256 + 128 * half, 128)],
                             ssems[ln])

        def wait_store(ln):
            pltpu.make_async_copy(rows_v.at[ln], t2_hbm.at[pl.ds(0, 128)],
                                  ssems[ln]).wait()

        start_load(0, 0)
        start_load(1, 1)

        def pair_body(p, carry):
            for ln in range(2):
                t = 2 * p + ln

                @pl.when(active(t))
                def _(t=t, ln=ln):
                    wait_load(ln)
                    for half in range(2):
                        if ln == 0:
                            @pl.when(p > 0)
                            def _(half=half):
                                wait_store(half)
                        else:
                            wait_store(half)

                        transpose_block(col_v.at[ln], rows_v.at[half],
                                        2 * half, 0)
                        transpose_block(col_v.at[ln], rows_v.at[half],
                                        2 * half + 1, EMB_DIM)
                        start_store(t, half, half)

                @pl.when(jnp.logical_and(p + 1 < npair, active(t + 2)))
                def _(t=t, ln=ln):
                    start_load(t + 2, ln)

            return carry

        lax.fori_loop(0, npair, pair_body, 0)
        wait_store(0)
        wait_store(1)

        @pl.when(jnp.logical_and(wid == 0, tail > 0))
        def _():
            pltpu.sync_copy(tt_hbm.at[:, pl.ds(nblk * 128, tail)], tcol_v)

            def pbody(pp, c):
                r16 = (pp + iota16) & (tail // 2 - 1)
                col2 = 2 * r16
                for g in range(8):
                    vals = plsc.load_gather(tcol_v, [rsrc[g], col2 + cadd[g]])
                    plsc.store_scatter(trows_v, [r16, ccs[g]], vals)
                return c

            lax.fori_loop(0, tail // 2, pbody, 0)
            pltpu.sync_copy(trows_v, t2_hbm.at[pl.ds(nblk * 64, tail // 2)])

    return format_kernel


@functools.lru_cache(maxsize=None)
def _make_gather(S: int, B: int, V2: int):
    info = plsc.get_sparse_core_info()
    nw = info.num_cores * info.num_subcores
    assert B == 128 * nw and S % 4 == 0
    nq = S // 4
    mesh = plsc.VectorSubcoreMesh(core_axis_name="c", subcore_axis_name="s")

    @functools.partial(
        pl.kernel,
        mesh=mesh,
        compiler_params=_PARAMS,
        out_type=jax.ShapeDtypeStruct((S, EMB_DIM, B), jnp.float32),
        scratch_types=[
            pltpu.VMEM((S, 128), jnp.int32),
            pltpu.VMEM((4, 128), jnp.int32),
            pltpu.VMEM((4, 128, 128), jnp.float32),
            pltpu.VMEM((4, EMB_DIM, 128), jnp.float32),
            [pltpu.SemaphoreType.DMA] * 4,
            [pltpu.SemaphoreType.DMA] * 4,
        ],
    )
    def gather_kernel(xt_hbm, t2_hbm, out_hbm, idx_v, idx2_v, rows_v,
                      tiles_v, gsems, ssems):
        wid = lax.axis_index("s") * info.num_cores + lax.axis_index("c")
        boff = wid * 128
        iota16 = lax.iota(jnp.int32, 16)
        bbs = tuple(iota16 + 16 * g for g in range(8))

        pltpu.sync_copy(xt_hbm.at[:, pl.ds(boff, 128)], idx_v)

        def start_gather(s, ln):
            for g in range(8):
                idx2_v[ln, pl.ds(16 * g, 16)] = idx_v[s, pl.ds(16 * g, 16)] >> 1
            pltpu.async_copy(t2_hbm.at[idx2_v.at[ln]], rows_v.at[ln],
                             gsems[ln])

        def wait_gather(ln):
            pltpu.make_async_copy(t2_hbm.at[idx2_v.at[ln]], rows_v.at[ln],
                                  gsems[ln]).wait()

        def transpose(s, ln):
            rows = rows_v.at[ln]
            tiles = tiles_v.at[ln]
            bases = tuple(
                (idx_v[s, pl.ds(16 * g, 16)] & 1) * EMB_DIM for g in range(8))

            @functools.partial(plsc.parallel_loop, 0, EMB_DIM, unroll=4,
                               carry=bases)
            def dbody(dd, carry):
                d16 = (dd + iota16) & (EMB_DIM - 1)
                for g in range(8):
                    vals = plsc.load_gather(rows, [bbs[g], carry[g] + d16])
                    plsc.store_scatter(tiles, [d16, bbs[g]], vals)
                return carry

        def start_store(s, ln):
            pltpu.async_copy(tiles_v.at[ln],
                             out_hbm.at[s, :, pl.ds(boff, 128)], ssems[ln])

        def wait_store(ln):
            pltpu.make_async_copy(tiles_v.at[ln],
                                  out_hbm.at[0, :, pl.ds(boff, 128)],
                                  ssems[ln]).wait()

        for ln in range(4):
            start_gather(ln, ln)

        def quad_body(q, carry):
            for ln in range(4):
                s = 4 * q + ln

                @pl.when(q > 0)
                def _(ln=ln):
                    wait_store(ln)

                wait_gather(ln)
                transpose(s, ln)

                @pl.when(q + 1 < nq)
                def _(s=s, ln=ln):
                    start_gather(s + 4, ln)

                start_store(s, ln)
            return carry

        lax.fori_loop(0, nq, quad_body, 0)
        for ln in range(4):
            wait_store(ln)

    return gather_kernel


def kernel(x, table):
    b, s = x.shape
    v, d = table.shape
    t2 = _make_format(v)(table.T)
    out_t = _make_gather(s, b, v // 2)(x.T, t2)
    return out_t.transpose(2, 0, 1)
